# baseline (speedup 1.0000x reference)
import jax
import jax.numpy as jnp
from jax import lax
from jax.experimental import pallas as pl
from jax.experimental.pallas import tpu as pltpu
from jax.experimental.pallas import tpu_sc as plsc

_NUMEL = 16777216
_NNZ = 1677721
_LR = 0.1

_NC = 2
_NS = 16
_NW = _NC * _NS
_G = 4096
_GROUPS = 13
_NBUF = 5
_P = _G * _GROUPS
_TOTAL = _NW * _P


def _sc_body(idx_hbm, val_hbm, param_hbm, out_ref, *scr):
    idx_v = scr[0:_NBUF]
    val_v = scr[_NBUF:2 * _NBUF]
    gat_v = scr[2 * _NBUF:3 * _NBUF]
    sem_ld = scr[3 * _NBUF:4 * _NBUF]
    sem_g = scr[4 * _NBUF:5 * _NBUF]
    sem_s = scr[5 * _NBUF:6 * _NBUF]
    c = lax.axis_index("c")
    s = lax.axis_index("s")
    wid = s * _NC + c
    base0 = wid * _P

    def start_load(t, m):
        off = base0 + t * _G
        pltpu.make_async_copy(idx_hbm.at[pl.ds(off, _G)], idx_v[m],
                              sem_ld[m]).start()
        pltpu.make_async_copy(val_hbm.at[pl.ds(off, _G)], val_v[m],
                              sem_ld[m]).start()

    def wait_load(m):
        pltpu.make_async_copy(idx_hbm.at[pl.ds(0, _G)], idx_v[m],
                              sem_ld[m]).wait()
        pltpu.make_async_copy(val_hbm.at[pl.ds(0, _G)], val_v[m],
                              sem_ld[m]).wait()

    def fire_gather(m):
        pltpu.make_async_copy(param_hbm.at[idx_v[m]], gat_v[m],
                              sem_g[m]).start()

    def process(n):
        pltpu.make_async_copy(param_hbm.at[idx_v[n]], gat_v[n],
                              sem_g[n]).wait()

        @pl.loop(0, _G // 16, unroll=4)
        def _cmp(i):
            sl = pl.ds(i * 16, 16)
            gat_v[n][sl] = gat_v[n][sl] + _LR * val_v[n][sl]

        pltpu.make_async_copy(gat_v[n], out_ref.at[idx_v[n]],
                              sem_s[n]).start()

    def drain_scatter(m):
        pltpu.make_async_copy(gat_v[m], out_ref.at[idx_v[m]],
                              sem_s[m]).wait()

    for t in range(_GROUPS):
        m = t % _NBUF
        if t >= _NBUF:
            drain_scatter(m)
        start_load(t, m)
        if t >= 2:
            process((t - 2) % _NBUF)
        wait_load(m)
        fire_gather(m)
    for t in (_GROUPS - 2, _GROUPS - 1):
        process(t % _NBUF)
    for t in range(_GROUPS - _NBUF, _GROUPS):
        drain_scatter(t % _NBUF)


_sc_update = pl.kernel(
    _sc_body,
    out_type=(),
    mesh=plsc.VectorSubcoreMesh(core_axis_name="c", subcore_axis_name="s"),
    scratch_types=(
        [pltpu.VMEM((_G,), jnp.int32) for _ in range(_NBUF)]
        + [pltpu.VMEM((_G,), jnp.float32) for _ in range(_NBUF)]
        + [pltpu.VMEM((_G,), jnp.float32) for _ in range(_NBUF)]
        + [pltpu.SemaphoreType.DMA] * (3 * _NBUF)),
)


def kernel(param, values, indices):
    idx = indices.astype(jnp.int32)
    pad = _TOTAL - _NNZ
    idxp = jnp.pad(idx, (0, pad), mode="wrap")
    valp = jnp.pad(values, (0, pad), mode="wrap")
    out_ref = jax.new_ref(param)
    _sc_update(idxp, valp, param, out_ref)
    return out_ref[...]
